# Initial kernel scaffold; baseline (speedup 1.0000x reference)
import jax
import jax.numpy as jnp
from jax.experimental import pallas as pl


def kernel(x, index, weights):
    raise NotImplementedError("write your pallas kernel here")


# speedup vs baseline: 2.2952x; 2.2952x over previous
import jax
import jax.numpy as jnp
from jax import lax
from jax.experimental import pallas as pl
from jax.experimental.pallas import tpu as pltpu
from jax.experimental.pallas import tpu_sc as plsc

N_EDGES = 320000
N_SEG = 10000
N_SEG_PAD = 10240
D = 128
DH = D // 2

NC = 2
NS = 16

CHUNK = 512
ROWS_PER_CHUNK = CHUNK // 128
N_CHUNKS = N_EDGES // CHUNK
ITERS = (N_CHUNKS + NS - 1) // NS

SEG_PER_SUB = N_SEG_PAD // NS
ZROWS = 128


def _sc_body(xh_hbm, idx_hbm, w_hbm, psums_hbm, pcnts_hbm,
             xbuf, ridbuf, idxbuf, wbuf, onesbuf, zrow, zcnt, acc_s, acc_c):
    c = lax.axis_index("c")
    s = lax.axis_index("s")

    zeros16 = jnp.zeros((16,), jnp.float32)
    iota16 = lax.iota(jnp.int32, 16)
    onevec = jnp.where(iota16 == 0, 1.0, 0.0).astype(jnp.float32)

    def fill_z(i, carry):
        for j in range(DH // 16):
            zrow[i, pl.ds(j * 16, 16)] = zeros16
        zcnt[i, :] = zeros16
        return carry

    lax.fori_loop(0, ZROWS, fill_z, 0)

    def fill_ones(i, carry):
        onesbuf[i, :] = onevec
        return carry

    lax.fori_loop(0, 128, fill_ones, 0)

    for k in range(SEG_PER_SUB // ZROWS):
        off = s * SEG_PER_SUB + k * ZROWS
        pltpu.sync_copy(zrow, acc_s.at[pl.ds(off, ZROWS)])
        pltpu.sync_copy(zcnt, acc_c.at[pl.ds(off, ZROWS)])
    plsc.subcore_barrier()

    def chunk_body(i, carry):
        t = s + i * NS

        @pl.when(t < N_CHUNKS)
        def _():
            base = t * CHUNK
            rowb = t * ROWS_PER_CHUNK
            pltpu.sync_copy(idx_hbm.at[pl.ds(rowb, ROWS_PER_CHUNK)], idxbuf)
            pltpu.sync_copy(w_hbm.at[pl.ds(rowb, ROWS_PER_CHUNK)], wbuf)

            def rid_body(g, gc):
                e0 = base + g * 16
                ridbuf[g // 8, pl.ds((g % 8) * 16, 16)] = 2 * (e0 + iota16) + c
                return gc

            lax.fori_loop(0, CHUNK // 16, rid_body, 0)

            for j in range(ROWS_PER_CHUNK):
                pltpu.sync_copy(xh_hbm.at[ridbuf.at[j]],
                                xbuf.at[pl.ds(j * 128, 128)])

            def group_body(g, gc):
                wvec = wbuf[g // 8, pl.ds((g % 8) * 16, 16)]
                for l in range(16):
                    wscal = wvec[l]
                    e = g * 16 + l
                    for j in range(DH // 16):
                        xbuf[e, pl.ds(j * 16, 16)] = xbuf[e, pl.ds(j * 16, 16)] * wscal
                return gc

            lax.fori_loop(0, CHUNK // 16, group_body, 0)

            for j in range(ROWS_PER_CHUNK):
                pltpu.sync_copy(xbuf.at[pl.ds(j * 128, 128)],
                                acc_s.at[idxbuf.at[j]], add=True)

            @pl.when(c == 0)
            def _():
                for j in range(ROWS_PER_CHUNK):
                    pltpu.sync_copy(onesbuf, acc_c.at[idxbuf.at[j]], add=True)

        return carry

    lax.fori_loop(0, ITERS, chunk_body, 0)
    plsc.subcore_barrier()

    for k in range(SEG_PER_SUB // ZROWS):
        off = s * SEG_PER_SUB + k * ZROWS
        hoff = c * N_SEG_PAD + off
        pltpu.sync_copy(acc_s.at[pl.ds(off, ZROWS)], psums_hbm.at[pl.ds(hoff, ZROWS)])

        @pl.when(c == 0)
        def _():
            pltpu.sync_copy(acc_c.at[pl.ds(off, ZROWS)], pcnts_hbm.at[pl.ds(off, ZROWS)])


def _combine_body(p0, p1, cn, o):
    cnt = jnp.sum(cn[...], axis=1)
    cnt = jnp.maximum(cnt, 1.0)
    o[...] = jnp.concatenate([p0[...], p1[...]], axis=1) / cnt[:, None]


BLK = 640


def kernel(x, index, weights):
    xh = x.reshape(2 * N_EDGES, DH)
    idx2d = index.reshape(N_EDGES // 128, 128)
    w2d = weights.reshape(N_EDGES // 128, 128)

    mesh = plsc.VectorSubcoreMesh(core_axis_name="c", subcore_axis_name="s")
    phase1 = pl.kernel(
        _sc_body,
        out_type=[
            jax.ShapeDtypeStruct((NC * N_SEG_PAD, DH), jnp.float32),
            jax.ShapeDtypeStruct((N_SEG_PAD, 16), jnp.float32),
        ],
        mesh=mesh,
        compiler_params=pltpu.CompilerParams(use_tc_tiling_on_sc=False),
        scratch_types=[
            pltpu.VMEM((CHUNK, DH), jnp.float32),
            pltpu.VMEM((ROWS_PER_CHUNK, 128), jnp.int32),
            pltpu.VMEM((ROWS_PER_CHUNK, 128), jnp.int32),
            pltpu.VMEM((ROWS_PER_CHUNK, 128), jnp.float32),
            pltpu.VMEM((128, 16), jnp.float32),
            pltpu.VMEM((ZROWS, DH), jnp.float32),
            pltpu.VMEM((ZROWS, 16), jnp.float32),
            pltpu.VMEM_SHARED((N_SEG_PAD, DH), jnp.float32),
            pltpu.VMEM_SHARED((N_SEG_PAD, 16), jnp.float32),
        ],
    )
    psums, pcnts = phase1(xh, idx2d, w2d)

    nblk = N_SEG_PAD // BLK
    out = pl.pallas_call(
        _combine_body,
        grid=(nblk,),
        in_specs=[
            pl.BlockSpec((BLK, DH), lambda i: (i, 0)),
            pl.BlockSpec((BLK, DH), lambda i: (i + nblk, 0)),
            pl.BlockSpec((BLK, 16), lambda i: (i, 0)),
        ],
        out_specs=pl.BlockSpec((BLK, D), lambda i: (i, 0)),
        out_shape=jax.ShapeDtypeStruct((N_SEG_PAD, D), jnp.float32),
    )(psums, psums, pcnts)
    return out[:N_SEG]


# speedup vs baseline: 3.3346x; 1.4529x over previous
import jax
import jax.numpy as jnp
from jax import lax
from jax.experimental import pallas as pl
from jax.experimental.pallas import tpu as pltpu
from jax.experimental.pallas import tpu_sc as plsc

N_EDGES = 320000
N_SEG = 10000
N_SEG_PAD = 10112
D = 128
DH = D // 2

NC = 2
NS = 16

CHUNK = 512
ROWS_PER_CHUNK = CHUNK // 128
N_CHUNKS = N_EDGES // CHUNK
ITERS = (N_CHUNKS + NS - 1) // NS

SEG_PER_SUB = N_SEG_PAD // NS
CL = 8


def _sc_body(xh_hbm, idx_hbm, w_hbm, zs_hbm, zc_hbm, ones_hbm,
             psums_hbm, pcnts_hbm,
             xbufs, ridbufs, idxbufs, wbufs, onesbuf,
             acc_s, acc_c, gsems, ssems):
    c = lax.axis_index("c")
    s = lax.axis_index("s")

    iota16 = lax.iota(jnp.int32, 16)

    pltpu.sync_copy(ones_hbm, onesbuf)
    off = s * SEG_PER_SUB
    pltpu.sync_copy(zs_hbm, acc_s.at[pl.ds(off, SEG_PER_SUB)])
    pltpu.sync_copy(zc_hbm, acc_c.at[pl.ds(off, SEG_PER_SUB)])
    plsc.subcore_barrier()

    def chunk_t(i):
        return s + i * NS


    def issue_gathers(i, b):
        t = chunk_t(i)
        base = t * CHUNK
        rowb = t * ROWS_PER_CHUNK
        xbuf, ridbuf, idxbuf, wbuf = xbufs[b], ridbufs[b], idxbufs[b], wbufs[b]

        def rid_body(g, gc):
            e0 = base + g * 16
            ridbuf[g // 8, pl.ds((g % 8) * 16, 16)] = 2 * (e0 + iota16) + c
            return gc

        lax.fori_loop(0, CHUNK // 16, rid_body, 0)

        pltpu.async_copy(idx_hbm.at[pl.ds(rowb, ROWS_PER_CHUNK)], idxbuf, gsems[b])
        pltpu.async_copy(w_hbm.at[pl.ds(rowb, ROWS_PER_CHUNK)], wbuf, gsems[b])
        for j in range(ROWS_PER_CHUNK):
            pltpu.async_copy(xh_hbm.at[ridbuf.at[j]],
                             xbuf.at[pl.ds(j * 128, 128)], gsems[b])

    def wait_gathers(i, b):
        t = chunk_t(i)
        rowb = t * ROWS_PER_CHUNK
        xbuf, ridbuf, idxbuf, wbuf = xbufs[b], ridbufs[b], idxbufs[b], wbufs[b]
        pltpu.make_async_copy(idx_hbm.at[pl.ds(rowb, ROWS_PER_CHUNK)], idxbuf,
                              gsems[b]).wait()
        pltpu.make_async_copy(w_hbm.at[pl.ds(rowb, ROWS_PER_CHUNK)], wbuf,
                              gsems[b]).wait()
        for j in range(ROWS_PER_CHUNK):
            pltpu.make_async_copy(xh_hbm.at[ridbuf.at[j]],
                                  xbuf.at[pl.ds(j * 128, 128)], gsems[b]).wait()

    def process_chunk(b):
        xbuf, idxbuf, wbuf = xbufs[b], idxbufs[b], wbufs[b]

        def group_body(g, gc):
            wvec = wbuf[g // 8, pl.ds((g % 8) * 16, 16)]
            for l in range(16):
                wscal = wvec[l]
                e = g * 16 + l
                for j in range(DH // 16):
                    xbuf[e, pl.ds(j * 16, 16)] = xbuf[e, pl.ds(j * 16, 16)] * wscal
            return gc

        lax.fori_loop(0, CHUNK // 16, group_body, 0)

        for j in range(ROWS_PER_CHUNK):
            pltpu.async_copy(xbuf.at[pl.ds(j * 128, 128)],
                             acc_s.at[idxbuf.at[j]], ssems[b], add=True)

        @pl.when(c == 0)
        def _():
            for j in range(ROWS_PER_CHUNK):
                pltpu.async_copy(onesbuf, acc_c.at[idxbuf.at[j]], ssems[b],
                                 add=True)

    def drain_scatters(b):
        xbuf, idxbuf = xbufs[b], idxbufs[b]
        for j in range(ROWS_PER_CHUNK):
            pltpu.make_async_copy(xbuf.at[pl.ds(j * 128, 128)],
                                  acc_s.at[idxbuf.at[j]], ssems[b]).wait()

        @pl.when(c == 0)
        def _():
            for j in range(ROWS_PER_CHUNK):
                pltpu.make_async_copy(onesbuf, acc_c.at[idxbuf.at[j]],
                                      ssems[b]).wait()


    def loop_body(jj, carry):
        for b in range(2):
            i = 2 * jj + b
            t = chunk_t(i)

            @pl.when(t < N_CHUNKS)
            def _():
                @pl.when(t >= s + 2 * NS)
                def _():
                    drain_scatters(b)

                issue_gathers(i, b)

            @pl.when((i >= 1) & (t - NS < N_CHUNKS))
            def _():
                wait_gathers(i - 1, 1 - b)
                process_chunk(1 - b)

        return carry

    lax.fori_loop(0, ITERS // 2, loop_body, 0)

    t_last = chunk_t(ITERS - 1)

    @pl.when(t_last < N_CHUNKS)
    def _():
        wait_gathers(ITERS - 1, 1)
        process_chunk(1)
        drain_scatters(1)

    @pl.when(chunk_t(ITERS - 2) < N_CHUNKS)
    def _():
        drain_scatters(0)

    plsc.subcore_barrier()

    hoff = c * N_SEG_PAD + off
    pltpu.sync_copy(acc_s.at[pl.ds(off, SEG_PER_SUB)],
                    psums_hbm.at[pl.ds(hoff, SEG_PER_SUB)])

    @pl.when(c == 0)
    def _():
        pltpu.sync_copy(acc_c.at[pl.ds(off, SEG_PER_SUB)],
                        pcnts_hbm.at[pl.ds(off, SEG_PER_SUB)])


def _combine_body(p0, p1, cn, o):
    cnt = jnp.sum(cn[...], axis=1)
    cnt = jnp.maximum(cnt, 1.0)
    o[...] = jnp.concatenate([p0[...], p1[...]], axis=1) / cnt[:, None]


BLK = 632


def kernel(x, index, weights):
    xh = x.reshape(2 * N_EDGES, DH)
    idx2d = index.reshape(N_EDGES // 128, 128)
    w2d = weights.reshape(N_EDGES // 128, 128)

    mesh = plsc.VectorSubcoreMesh(core_axis_name="c", subcore_axis_name="s")
    phase1 = pl.kernel(
        _sc_body,
        out_type=[
            jax.ShapeDtypeStruct((NC * N_SEG_PAD, DH), jnp.float32),
            jax.ShapeDtypeStruct((N_SEG_PAD, CL), jnp.float32),
        ],
        mesh=mesh,
        compiler_params=pltpu.CompilerParams(use_tc_tiling_on_sc=False),
        scratch_types=[
            [pltpu.VMEM((CHUNK, DH), jnp.float32) for _ in range(2)],
            [pltpu.VMEM((ROWS_PER_CHUNK, 128), jnp.int32) for _ in range(2)],
            [pltpu.VMEM((ROWS_PER_CHUNK, 128), jnp.int32) for _ in range(2)],
            [pltpu.VMEM((ROWS_PER_CHUNK, 128), jnp.float32) for _ in range(2)],
            pltpu.VMEM((128, CL), jnp.float32),
            pltpu.VMEM_SHARED((N_SEG_PAD, DH), jnp.float32),
            pltpu.VMEM_SHARED((N_SEG_PAD, CL), jnp.float32),
            [pltpu.SemaphoreType.DMA for _ in range(2)],
            [pltpu.SemaphoreType.DMA for _ in range(2)],
        ],
    )
    zs = jnp.zeros((SEG_PER_SUB, DH), jnp.float32)
    zc = jnp.zeros((SEG_PER_SUB, CL), jnp.float32)
    ones = jnp.zeros((128, CL), jnp.float32).at[:, 0].set(1.0)
    psums, pcnts = phase1(xh, idx2d, w2d, zs, zc, ones)

    nblk = N_SEG_PAD // BLK
    out = pl.pallas_call(
        _combine_body,
        grid=(nblk,),
        in_specs=[
            pl.BlockSpec((BLK, DH), lambda i: (i, 0)),
            pl.BlockSpec((BLK, DH), lambda i: (i + nblk, 0)),
            pl.BlockSpec((BLK, CL), lambda i: (i, 0)),
        ],
        out_specs=pl.BlockSpec((BLK, D), lambda i: (i, 0)),
        out_shape=jax.ShapeDtypeStruct((N_SEG_PAD, D), jnp.float32),
    )(psums, psums, pcnts)
    return out[:N_SEG]


# speedup vs baseline: 5.7883x; 1.7359x over previous
import jax
import jax.numpy as jnp
from jax import lax
from jax.experimental import pallas as pl
from jax.experimental.pallas import tpu as pltpu
from jax.experimental.pallas import tpu_sc as plsc

N_EDGES = 320000
N_SEG = 10000
N_SEG_PAD = 10112
D = 128
DH = D // 2

NC = 2
NS = 16

CHUNK = 256
ROWS_PER_CHUNK = CHUNK // 128
N_CHUNKS = N_EDGES // CHUNK
ITERS = (N_CHUNKS + NS - 1) // NS

SEG_PER_SUB = N_SEG_PAD // NS
CL = 8


def _sc_body(xh_hbm, idx_hbm, w_hbm, zs_hbm, zc_hbm, ones_hbm,
             psums_hbm, pcnts_hbm,
             xbufs, ridbufs, idxbufs, wbufs, obuf, onesbuf,
             acc_s, acc_c, gsems, ssem):
    c = lax.axis_index("c")
    s = lax.axis_index("s")

    iota16 = lax.iota(jnp.int32, 16)

    pltpu.sync_copy(ones_hbm, onesbuf)
    off = s * SEG_PER_SUB
    pltpu.sync_copy(zs_hbm, acc_s.at[pl.ds(off, SEG_PER_SUB)])
    pltpu.sync_copy(zc_hbm, acc_c.at[pl.ds(off, SEG_PER_SUB)])
    plsc.subcore_barrier()

    def chunk_t(i):
        return s + i * NS


    def issue_gathers(i, b):
        t = chunk_t(i)
        base = t * CHUNK
        rowb = t * ROWS_PER_CHUNK
        xbuf, ridbuf, idxbuf, wbuf = xbufs[b], ridbufs[b], idxbufs[b], wbufs[b]

        def rid_body(g, gc):
            e0 = base + g * 16
            ridbuf[g // 8, pl.ds((g % 8) * 16, 16)] = 2 * (e0 + iota16) + c
            return gc

        lax.fori_loop(0, CHUNK // 16, rid_body, 0)

        pltpu.async_copy(idx_hbm.at[pl.ds(rowb, ROWS_PER_CHUNK)], idxbuf, gsems[b])
        pltpu.async_copy(w_hbm.at[pl.ds(rowb, ROWS_PER_CHUNK)], wbuf, gsems[b])
        for j in range(ROWS_PER_CHUNK):
            pltpu.async_copy(xh_hbm.at[ridbuf.at[j]],
                             xbuf.at[pl.ds(j * 128, 128)], gsems[b])

    def wait_gathers(i, b):
        t = chunk_t(i)
        rowb = t * ROWS_PER_CHUNK
        xbuf, ridbuf, idxbuf, wbuf = xbufs[b], ridbufs[b], idxbufs[b], wbufs[b]
        pltpu.make_async_copy(idx_hbm.at[pl.ds(rowb, ROWS_PER_CHUNK)], idxbuf,
                              gsems[b]).wait()
        pltpu.make_async_copy(w_hbm.at[pl.ds(rowb, ROWS_PER_CHUNK)], wbuf,
                              gsems[b]).wait()
        for j in range(ROWS_PER_CHUNK):
            pltpu.make_async_copy(xh_hbm.at[ridbuf.at[j]],
                                  xbuf.at[pl.ds(j * 128, 128)], gsems[b]).wait()

    def process_chunk(b):
        xbuf, idxbuf, wbuf = xbufs[b], idxbufs[b], wbufs[b]

        def group_body(g, gc):
            wvec = wbuf[g // 8, pl.ds((g % 8) * 16, 16)]
            for l in range(16):
                wscal = wvec[l]
                e = g * 16 + l
                vals = [xbuf[e, pl.ds(j * 16, 16)] for j in range(DH // 16)]
                for j in range(DH // 16):
                    obuf[e, pl.ds(j * 16, 16)] = vals[j] * wscal
            return gc

        lax.fori_loop(0, CHUNK // 16, group_body, 0)

        for j in range(ROWS_PER_CHUNK):
            pltpu.async_copy(obuf.at[pl.ds(j * 128, 128)],
                             acc_s.at[idxbuf.at[j]], ssem, add=True)

        @pl.when(c == 0)
        def _():
            for j in range(ROWS_PER_CHUNK):
                pltpu.async_copy(onesbuf, acc_c.at[idxbuf.at[j]], ssem,
                                 add=True)

    def drain_scatters(b):
        idxbuf = idxbufs[b]
        for j in range(ROWS_PER_CHUNK):
            pltpu.make_async_copy(obuf.at[pl.ds(j * 128, 128)],
                                  acc_s.at[idxbuf.at[j]], ssem).wait()

        @pl.when(c == 0)
        def _():
            for j in range(ROWS_PER_CHUNK):
                pltpu.make_async_copy(onesbuf, acc_c.at[idxbuf.at[j]],
                                      ssem).wait()


    def loop_body(jj, carry):
        for b in range(2):
            i = 2 * jj + b
            t = chunk_t(i)

            @pl.when((t >= s + 2 * NS) & (t - 2 * NS < N_CHUNKS))
            def _():
                drain_scatters(b)

            @pl.when(t < N_CHUNKS)
            def _():
                issue_gathers(i, b)

            @pl.when((i >= 1) & (t - NS < N_CHUNKS))
            def _():
                wait_gathers(i - 1, 1 - b)
                process_chunk(1 - b)

        return carry

    lax.fori_loop(0, ITERS // 2 + 1, loop_body, 0)

    if ITERS % 2 == 1:
        @pl.when(chunk_t(ITERS - 1) < N_CHUNKS)
        def _():
            drain_scatters((ITERS - 1) % 2)

    plsc.subcore_barrier()

    hoff = c * N_SEG_PAD + off
    pltpu.sync_copy(acc_s.at[pl.ds(off, SEG_PER_SUB)],
                    psums_hbm.at[pl.ds(hoff, SEG_PER_SUB)])

    @pl.when(c == 0)
    def _():
        pltpu.sync_copy(acc_c.at[pl.ds(off, SEG_PER_SUB)],
                        pcnts_hbm.at[pl.ds(off, SEG_PER_SUB)])


def _combine_body(p0, p1, cn, o):
    cnt = jnp.sum(cn[...], axis=1)
    cnt = jnp.maximum(cnt, 1.0)
    o[...] = jnp.concatenate([p0[...], p1[...]], axis=1) / cnt[:, None]


BLK = 632


def kernel(x, index, weights):
    xh = x.reshape(2 * N_EDGES, DH)
    idx2d = index.reshape(N_EDGES // 128, 128)
    w2d = weights.reshape(N_EDGES // 128, 128)

    mesh = plsc.VectorSubcoreMesh(core_axis_name="c", subcore_axis_name="s")
    phase1 = pl.kernel(
        _sc_body,
        out_type=[
            jax.ShapeDtypeStruct((NC * N_SEG_PAD, DH), jnp.float32),
            jax.ShapeDtypeStruct((N_SEG_PAD, CL), jnp.float32),
        ],
        mesh=mesh,
        compiler_params=pltpu.CompilerParams(use_tc_tiling_on_sc=False),
        scratch_types=[
            [pltpu.VMEM((CHUNK, DH), jnp.float32) for _ in range(2)],
            [pltpu.VMEM((ROWS_PER_CHUNK, 128), jnp.int32) for _ in range(2)],
            [pltpu.VMEM((ROWS_PER_CHUNK, 128), jnp.int32) for _ in range(2)],
            [pltpu.VMEM((ROWS_PER_CHUNK, 128), jnp.float32) for _ in range(2)],
            pltpu.VMEM((CHUNK, DH), jnp.float32),
            pltpu.VMEM((128, CL), jnp.float32),
            pltpu.VMEM_SHARED((N_SEG_PAD, DH), jnp.float32),
            pltpu.VMEM_SHARED((N_SEG_PAD, CL), jnp.float32),
            [pltpu.SemaphoreType.DMA for _ in range(2)],
            pltpu.SemaphoreType.DMA,
        ],
    )
    zs = jnp.zeros((SEG_PER_SUB, DH), jnp.float32)
    zc = jnp.zeros((SEG_PER_SUB, CL), jnp.float32)
    ones = jnp.zeros((128, CL), jnp.float32).at[:, 0].set(1.0)
    psums, pcnts = phase1(xh, idx2d, w2d, zs, zc, ones)

    nblk = N_SEG_PAD // BLK
    out = pl.pallas_call(
        _combine_body,
        grid=(nblk,),
        in_specs=[
            pl.BlockSpec((BLK, DH), lambda i: (i, 0)),
            pl.BlockSpec((BLK, DH), lambda i: (i + nblk, 0)),
            pl.BlockSpec((BLK, CL), lambda i: (i, 0)),
        ],
        out_specs=pl.BlockSpec((BLK, D), lambda i: (i, 0)),
        out_shape=jax.ShapeDtypeStruct((N_SEG_PAD, D), jnp.float32),
    )(psums, psums, pcnts)
    return out[:N_SEG]


# speedup vs baseline: 7.2664x; 1.2554x over previous
import jax
import jax.numpy as jnp
from jax import lax
from jax.experimental import pallas as pl
from jax.experimental.pallas import tpu as pltpu
from jax.experimental.pallas import tpu_sc as plsc

N_EDGES = 320000
N_SEG = 10000
N_SEG_PAD = 10112
D = 128
DH = D // 2

NC = 2
NS = 16

CHUNK = 256
ROWS_PER_CHUNK = CHUNK // 128
N_CHUNKS = N_EDGES // CHUNK
ITERS = (N_CHUNKS + NS - 1) // NS

SEG_PER_SUB = N_SEG_PAD // NS
CL = 8


def _sc_body(xh_hbm, idx_hbm, w_hbm, zs_hbm, zc_hbm, ones_hbm,
             psums_hbm, pcnts_hbm,
             xbufs, ridbufs, idxbufs, wbufs, obufs, onesbuf,
             acc_s, acc_c, gsems, ssems):
    c = lax.axis_index("c")
    s = lax.axis_index("s")

    iota16 = lax.iota(jnp.int32, 16)

    pltpu.sync_copy(ones_hbm, onesbuf)
    off = s * SEG_PER_SUB
    pltpu.sync_copy(zs_hbm, acc_s.at[pl.ds(off, SEG_PER_SUB)])
    pltpu.sync_copy(zc_hbm, acc_c.at[pl.ds(off, SEG_PER_SUB)])
    plsc.subcore_barrier()

    def chunk_t(i):
        return s + i * NS


    def issue_gathers(i, xb, ib):
        t = chunk_t(i)
        base = t * CHUNK
        rowb = t * ROWS_PER_CHUNK
        xbuf, ridbuf, idxbuf, wbuf = xbufs[xb], ridbufs[xb], idxbufs[ib], wbufs[xb]

        def rid_body(g, gc):
            e0 = base + g * 16
            ridbuf[g // 8, pl.ds((g % 8) * 16, 16)] = 2 * (e0 + iota16) + c
            return gc

        lax.fori_loop(0, CHUNK // 16, rid_body, 0)

        pltpu.async_copy(idx_hbm.at[pl.ds(rowb, ROWS_PER_CHUNK)], idxbuf, gsems[xb])
        pltpu.async_copy(w_hbm.at[pl.ds(rowb, ROWS_PER_CHUNK)], wbuf, gsems[xb])
        for j in range(ROWS_PER_CHUNK):
            pltpu.async_copy(xh_hbm.at[ridbuf.at[j]],
                             xbuf.at[pl.ds(j * 128, 128)], gsems[xb])

    def wait_gathers(i, xb, ib):
        t = chunk_t(i)
        rowb = t * ROWS_PER_CHUNK
        xbuf, ridbuf, idxbuf, wbuf = xbufs[xb], ridbufs[xb], idxbufs[ib], wbufs[xb]
        pltpu.make_async_copy(idx_hbm.at[pl.ds(rowb, ROWS_PER_CHUNK)], idxbuf,
                              gsems[xb]).wait()
        pltpu.make_async_copy(w_hbm.at[pl.ds(rowb, ROWS_PER_CHUNK)], wbuf,
                              gsems[xb]).wait()
        for j in range(ROWS_PER_CHUNK):
            pltpu.make_async_copy(xh_hbm.at[ridbuf.at[j]],
                                  xbuf.at[pl.ds(j * 128, 128)], gsems[xb]).wait()

    def process_chunk(xb, ib):
        xbuf, idxbuf, wbuf, obuf = xbufs[xb], idxbufs[ib], wbufs[xb], obufs[xb]

        def group_body(g, gc):
            wvec = wbuf[g // 8, pl.ds((g % 8) * 16, 16)]
            for l in range(16):
                wscal = wvec[l]
                e = g * 16 + l
                vals = [xbuf[e, pl.ds(j * 16, 16)] for j in range(DH // 16)]
                for j in range(DH // 16):
                    obuf[e, pl.ds(j * 16, 16)] = vals[j] * wscal
            return gc

        lax.fori_loop(0, CHUNK // 16, group_body, 0)

        for j in range(ROWS_PER_CHUNK):
            pltpu.async_copy(obuf.at[pl.ds(j * 128, 128)],
                             acc_s.at[idxbuf.at[j]], ssems[xb], add=True)

        @pl.when(c == 0)
        def _():
            for j in range(ROWS_PER_CHUNK):
                pltpu.async_copy(onesbuf, acc_c.at[idxbuf.at[j]], ssems[xb],
                                 add=True)

    def drain_scatters(xb, ib):
        idxbuf, obuf = idxbufs[ib], obufs[xb]
        for j in range(ROWS_PER_CHUNK):
            pltpu.make_async_copy(obuf.at[pl.ds(j * 128, 128)],
                                  acc_s.at[idxbuf.at[j]], ssems[xb]).wait()

        @pl.when(c == 0)
        def _():
            for j in range(ROWS_PER_CHUNK):
                pltpu.make_async_copy(onesbuf, acc_c.at[idxbuf.at[j]],
                                      ssems[xb]).wait()


    def loop_body(jj, carry):
        for q in range(4):
            i = 4 * jj + q
            t = chunk_t(i)

            @pl.when((t >= s + 3 * NS) & (t - 3 * NS < N_CHUNKS))
            def _():
                drain_scatters((q + 1) % 2, (q + 1) % 4)

            @pl.when(t < N_CHUNKS)
            def _():
                issue_gathers(i, q % 2, q)

            @pl.when((i >= 1) & (t - NS < N_CHUNKS))
            def _():
                wait_gathers(i - 1, (q + 1) % 2, (q + 3) % 4)
                process_chunk((q + 1) % 2, (q + 3) % 4)

        return carry

    H4 = ITERS // 4 + 1
    lax.fori_loop(0, H4, loop_body, 0)

    for k in (4 * H4 - 3, 4 * H4 - 2):
        @pl.when(chunk_t(k) < N_CHUNKS)
        def _():
            drain_scatters(k % 2, k % 4)

    plsc.subcore_barrier()

    hoff = c * N_SEG_PAD + off
    pltpu.sync_copy(acc_s.at[pl.ds(off, SEG_PER_SUB)],
                    psums_hbm.at[pl.ds(hoff, SEG_PER_SUB)])

    @pl.when(c == 0)
    def _():
        pltpu.sync_copy(acc_c.at[pl.ds(off, SEG_PER_SUB)],
                        pcnts_hbm.at[pl.ds(off, SEG_PER_SUB)])


def _combine_body(p0, p1, cn, o):
    cnt = jnp.sum(cn[...], axis=1)
    cnt = jnp.maximum(cnt, 1.0)
    o[...] = jnp.concatenate([p0[...], p1[...]], axis=1) / cnt[:, None]


BLK = 632


def kernel(x, index, weights):
    xh = x.reshape(2 * N_EDGES, DH)
    idx2d = index.reshape(N_EDGES // 128, 128)
    w2d = weights.reshape(N_EDGES // 128, 128)

    mesh = plsc.VectorSubcoreMesh(core_axis_name="c", subcore_axis_name="s")
    phase1 = pl.kernel(
        _sc_body,
        out_type=[
            jax.ShapeDtypeStruct((NC * N_SEG_PAD, DH), jnp.float32),
            jax.ShapeDtypeStruct((N_SEG_PAD, CL), jnp.float32),
        ],
        mesh=mesh,
        compiler_params=pltpu.CompilerParams(use_tc_tiling_on_sc=False),
        scratch_types=[
            [pltpu.VMEM((CHUNK, DH), jnp.float32) for _ in range(2)],
            [pltpu.VMEM((ROWS_PER_CHUNK, 128), jnp.int32) for _ in range(2)],
            [pltpu.VMEM((ROWS_PER_CHUNK, 128), jnp.int32) for _ in range(4)],
            [pltpu.VMEM((ROWS_PER_CHUNK, 128), jnp.float32) for _ in range(2)],
            [pltpu.VMEM((CHUNK, DH), jnp.float32) for _ in range(2)],
            pltpu.VMEM((128, CL), jnp.float32),
            pltpu.VMEM_SHARED((N_SEG_PAD, DH), jnp.float32),
            pltpu.VMEM_SHARED((N_SEG_PAD, CL), jnp.float32),
            [pltpu.SemaphoreType.DMA for _ in range(2)],
            [pltpu.SemaphoreType.DMA for _ in range(2)],
        ],
    )
    zs = jnp.zeros((SEG_PER_SUB, DH), jnp.float32)
    zc = jnp.zeros((SEG_PER_SUB, CL), jnp.float32)
    ones = jnp.zeros((128, CL), jnp.float32).at[:, 0].set(1.0)
    psums, pcnts = phase1(xh, idx2d, w2d, zs, zc, ones)

    nblk = N_SEG_PAD // BLK
    out = pl.pallas_call(
        _combine_body,
        grid=(nblk,),
        in_specs=[
            pl.BlockSpec((BLK, DH), lambda i: (i, 0)),
            pl.BlockSpec((BLK, DH), lambda i: (i + nblk, 0)),
            pl.BlockSpec((BLK, CL), lambda i: (i, 0)),
        ],
        out_specs=pl.BlockSpec((BLK, D), lambda i: (i, 0)),
        out_shape=jax.ShapeDtypeStruct((N_SEG_PAD, D), jnp.float32),
    )(psums, psums, pcnts)
    return out[:N_SEG]


# speedup vs baseline: 7.2943x; 1.0038x over previous
import jax
import jax.numpy as jnp
from jax import lax
from jax.experimental import pallas as pl
from jax.experimental.pallas import tpu as pltpu
from jax.experimental.pallas import tpu_sc as plsc

N_EDGES = 320000
N_SEG = 10000
N_SEG_PAD = 10112
D = 128
DH = D // 2

NC = 2
NS = 16

CHUNK = 256
ROWS_PER_CHUNK = CHUNK // 128
N_CHUNKS = N_EDGES // CHUNK
ITERS = (N_CHUNKS + NS - 1) // NS

SEG_PER_SUB = N_SEG_PAD // NS
CL = 8


def _sc_body(xh_hbm, idx_hbm, w_hbm, zs_hbm, zc_hbm, ones_hbm,
             psums_hbm, pcnts_hbm,
             xbufs, ridbufs, idxbufs, wbufs, obufs, onesbuf,
             acc_s, acc_c, gsems, ssems):
    c = lax.axis_index("c")
    s = lax.axis_index("s")

    iota16 = lax.iota(jnp.int32, 16)

    pltpu.sync_copy(ones_hbm, onesbuf)
    off = s * SEG_PER_SUB
    pltpu.sync_copy(zs_hbm, acc_s.at[pl.ds(off, SEG_PER_SUB)])
    pltpu.sync_copy(zc_hbm, acc_c.at[pl.ds(off, SEG_PER_SUB)])
    plsc.subcore_barrier()

    def chunk_t(i):
        return s + i * NS


    def issue_gathers(i, xb, ib):
        t = chunk_t(i)
        base = t * CHUNK
        rowb = t * ROWS_PER_CHUNK
        xbuf, ridbuf, idxbuf, wbuf = xbufs[xb], ridbufs[xb], idxbufs[ib], wbufs[xb]

        def rid_body(g, gc):
            e0 = base + g * 16
            ridbuf[g // 8, pl.ds((g % 8) * 16, 16)] = 2 * (e0 + iota16) + c
            return gc

        lax.fori_loop(0, CHUNK // 16, rid_body, 0)

        pltpu.async_copy(idx_hbm.at[pl.ds(rowb, ROWS_PER_CHUNK)], idxbuf, gsems[xb])
        pltpu.async_copy(w_hbm.at[pl.ds(rowb, ROWS_PER_CHUNK)], wbuf, gsems[xb])
        for j in range(ROWS_PER_CHUNK):
            pltpu.async_copy(xh_hbm.at[ridbuf.at[j]],
                             xbuf.at[pl.ds(j * 128, 128)], gsems[xb])

    def wait_gathers(i, xb, ib):
        t = chunk_t(i)
        rowb = t * ROWS_PER_CHUNK
        xbuf, ridbuf, idxbuf, wbuf = xbufs[xb], ridbufs[xb], idxbufs[ib], wbufs[xb]
        pltpu.make_async_copy(idx_hbm.at[pl.ds(rowb, ROWS_PER_CHUNK)], idxbuf,
                              gsems[xb]).wait()
        pltpu.make_async_copy(w_hbm.at[pl.ds(rowb, ROWS_PER_CHUNK)], wbuf,
                              gsems[xb]).wait()
        for j in range(ROWS_PER_CHUNK):
            pltpu.make_async_copy(xh_hbm.at[ridbuf.at[j]],
                                  xbuf.at[pl.ds(j * 128, 128)], gsems[xb]).wait()

    def process_chunk(xb, ib):
        xbuf, idxbuf, wbuf, obuf = xbufs[xb], idxbufs[ib], wbufs[xb], obufs[xb]

        def group_body(g, gc):
            wvec = wbuf[g // 8, pl.ds((g % 8) * 16, 16)]
            for l in range(0, 16, 2):
                e0 = g * 16 + l
                e1 = e0 + 1
                w0 = wvec[l]
                w1 = wvec[l + 1]
                vals0 = [xbuf[e0, pl.ds(j * 16, 16)] for j in range(DH // 16)]
                vals1 = [xbuf[e1, pl.ds(j * 16, 16)] for j in range(DH // 16)]
                for j in range(DH // 16):
                    obuf[e0, pl.ds(j * 16, 16)] = vals0[j] * w0
                for j in range(DH // 16):
                    obuf[e1, pl.ds(j * 16, 16)] = vals1[j] * w1
            return gc

        lax.fori_loop(0, CHUNK // 16, group_body, 0)

        for j in range(ROWS_PER_CHUNK):
            pltpu.async_copy(obuf.at[pl.ds(j * 128, 128)],
                             acc_s.at[idxbuf.at[j]], ssems[xb], add=True)

        @pl.when(c == 0)
        def _():
            for j in range(ROWS_PER_CHUNK):
                pltpu.async_copy(onesbuf, acc_c.at[idxbuf.at[j]], ssems[xb],
                                 add=True)

    def drain_scatters(xb, ib):
        idxbuf, obuf = idxbufs[ib], obufs[xb]
        for j in range(ROWS_PER_CHUNK):
            pltpu.make_async_copy(obuf.at[pl.ds(j * 128, 128)],
                                  acc_s.at[idxbuf.at[j]], ssems[xb]).wait()

        @pl.when(c == 0)
        def _():
            for j in range(ROWS_PER_CHUNK):
                pltpu.make_async_copy(onesbuf, acc_c.at[idxbuf.at[j]],
                                      ssems[xb]).wait()


    def loop_body(jj, carry):
        for q in range(4):
            i = 4 * jj + q
            t = chunk_t(i)

            @pl.when((t >= s + 3 * NS) & (t - 3 * NS < N_CHUNKS))
            def _():
                drain_scatters((q + 1) % 2, (q + 1) % 4)

            @pl.when(t < N_CHUNKS)
            def _():
                issue_gathers(i, q % 2, q)

            @pl.when((i >= 1) & (t - NS < N_CHUNKS))
            def _():
                wait_gathers(i - 1, (q + 1) % 2, (q + 3) % 4)
                process_chunk((q + 1) % 2, (q + 3) % 4)

        return carry

    H4 = ITERS // 4 + 1
    lax.fori_loop(0, H4, loop_body, 0)

    for k in (4 * H4 - 3, 4 * H4 - 2):
        @pl.when(chunk_t(k) < N_CHUNKS)
        def _():
            drain_scatters(k % 2, k % 4)

    plsc.subcore_barrier()

    hoff = c * N_SEG_PAD + off
    pltpu.sync_copy(acc_s.at[pl.ds(off, SEG_PER_SUB)],
                    psums_hbm.at[pl.ds(hoff, SEG_PER_SUB)])

    @pl.when(c == 0)
    def _():
        pltpu.sync_copy(acc_c.at[pl.ds(off, SEG_PER_SUB)],
                        pcnts_hbm.at[pl.ds(off, SEG_PER_SUB)])


def _combine_body(p0, p1, cn, o):
    cnt = jnp.sum(cn[...], axis=1)
    cnt = jnp.maximum(cnt, 1.0)
    o[...] = jnp.concatenate([p0[...], p1[...]], axis=1) / cnt[:, None]


BLK = 632


def kernel(x, index, weights):
    xh = x.reshape(2 * N_EDGES, DH)
    idx2d = index.reshape(N_EDGES // 128, 128)
    w2d = weights.reshape(N_EDGES // 128, 128)

    mesh = plsc.VectorSubcoreMesh(core_axis_name="c", subcore_axis_name="s")
    phase1 = pl.kernel(
        _sc_body,
        out_type=[
            jax.ShapeDtypeStruct((NC * N_SEG_PAD, DH), jnp.float32),
            jax.ShapeDtypeStruct((N_SEG_PAD, CL), jnp.float32),
        ],
        mesh=mesh,
        compiler_params=pltpu.CompilerParams(use_tc_tiling_on_sc=False),
        scratch_types=[
            [pltpu.VMEM((CHUNK, DH), jnp.float32) for _ in range(2)],
            [pltpu.VMEM((ROWS_PER_CHUNK, 128), jnp.int32) for _ in range(2)],
            [pltpu.VMEM((ROWS_PER_CHUNK, 128), jnp.int32) for _ in range(4)],
            [pltpu.VMEM((ROWS_PER_CHUNK, 128), jnp.float32) for _ in range(2)],
            [pltpu.VMEM((CHUNK, DH), jnp.float32) for _ in range(2)],
            pltpu.VMEM((128, CL), jnp.float32),
            pltpu.VMEM_SHARED((N_SEG_PAD, DH), jnp.float32),
            pltpu.VMEM_SHARED((N_SEG_PAD, CL), jnp.float32),
            [pltpu.SemaphoreType.DMA for _ in range(2)],
            [pltpu.SemaphoreType.DMA for _ in range(2)],
        ],
    )
    zs = jnp.zeros((SEG_PER_SUB, DH), jnp.float32)
    zc = jnp.zeros((SEG_PER_SUB, CL), jnp.float32)
    ones = jnp.zeros((128, CL), jnp.float32).at[:, 0].set(1.0)
    psums, pcnts = phase1(xh, idx2d, w2d, zs, zc, ones)

    nblk = N_SEG_PAD // BLK
    out = pl.pallas_call(
        _combine_body,
        grid=(nblk,),
        in_specs=[
            pl.BlockSpec((BLK, DH), lambda i: (i, 0)),
            pl.BlockSpec((BLK, DH), lambda i: (i + nblk, 0)),
            pl.BlockSpec((BLK, CL), lambda i: (i, 0)),
        ],
        out_specs=pl.BlockSpec((BLK, D), lambda i: (i, 0)),
        out_shape=jax.ShapeDtypeStruct((N_SEG_PAD, D), jnp.float32),
    )(psums, psums, pcnts)
    return out[:N_SEG]
